# baseline (speedup 1.0000x reference)
import functools
import math

import jax
import jax.numpy as jnp
from jax import lax
from jax.experimental import pallas as pl
from jax.experimental.pallas import tpu as pltpu
from jax.experimental.pallas import tpu_sc as plsc

DATA_LEN = 100000
RHO = 0.001
LR_DUAL = 0.001
MARGIN = 1.0
LAM = 1.0
LOG_RHO = math.log(RHO)

B = 4096
NC, NS = 2, 16
NW = NC * NS
PER_W = B // NW
CHUNK = 3136
PAD_LEN = NW * CHUNK

R_BLK = 256
NBR = B // R_BLK
C_BLK = 512
NCB = B // C_BLK

LOG2E = 1.4426950408889634
LN2 = 0.6931471805599453
SQRT_LOG2E = LOG2E ** 0.5
KEY_PAD = -(2 ** 30)

_MESH = dict(core_axis_name="c", subcore_axis_name="s", num_cores=NC,
             num_subcores=NS)


def _sc_compact_body(yt_hbm, psc_hbm, psr_hbm, key_hbm, idx_hbm, alpha_hbm,
                     pp_hbm, ap_hbm, kp_hbm, pn_hbm, cnt_hbm,
                     ytf_v, psc_v, psr_v, key_v, idx_v, a_v,
                     tgtp_v, tgtn_v, cnt_v, sem):
    wid = lax.axis_index("s") * NC + lax.axis_index("c")
    base = wid * PER_W
    d_yt = pltpu.async_copy(yt_hbm, ytf_v, sem)
    d_psc = pltpu.async_copy(psc_hbm.at[pl.ds(base, PER_W)], psc_v, sem)
    d_psr = pltpu.async_copy(psr_hbm.at[pl.ds(base, PER_W)], psr_v, sem)
    d_key = pltpu.async_copy(key_hbm.at[pl.ds(base, PER_W)], key_v, sem)
    d_idx = pltpu.async_copy(idx_hbm.at[pl.ds(base, PER_W)], idx_v, sem)
    d_idx.wait()
    d_ga = pltpu.async_copy(alpha_hbm.at[idx_v], a_v, sem)
    d_yt.wait()

    def pbody(k, s16):
        return s16 + ytf_v[pl.ds(k * 16, 16)]

    acc16 = lax.fori_loop(0, wid * 8, pbody, jnp.zeros((16,), jnp.float32))
    npos_before = jnp.sum(acc16).astype(jnp.int32)
    nneg_before = base - npos_before

    lanes = lax.iota(jnp.int32, 16)
    cpos = jnp.int32(0)
    cneg = jnp.int32(0)
    for v in range(PER_W // 16):
        yv = ytf_v[pl.ds(base + v * 16, 16)]
        posm = yv == 1.0
        pinc = posm.astype(jnp.int32)
        cs = plsc.cumsum(pinc)
        padbase = B + base + v * 16
        tgt_p = jnp.where(posm, npos_before + cpos + cs - 1, padbase + lanes)
        tgt_n = jnp.where(posm, padbase + lanes,
                          nneg_before + cneg + (lanes + 1 - cs) - 1)
        tgtp_v[pl.ds(v * 16, 16)] = tgt_p
        tgtn_v[pl.ds(v * 16, 16)] = tgt_n
        nposv = jnp.sum(pinc)
        cpos = cpos + nposv
        cneg = cneg + (16 - nposv)

    d_psc.wait()
    d_psr.wait()
    d_key.wait()
    d_ga.wait()
    d_pp = pltpu.async_copy(psc_v, pp_hbm.at[tgtp_v], sem)
    d_ap = pltpu.async_copy(a_v, ap_hbm.at[tgtp_v], sem)
    d_kp = pltpu.async_copy(key_v, kp_hbm.at[tgtp_v], sem)
    d_pn = pltpu.async_copy(psr_v, pn_hbm.at[tgtn_v], sem)
    d_pp.wait()
    d_ap.wait()
    d_kp.wait()
    d_pn.wait()

    @pl.when(wid == NW - 1)
    def _write_counts():
        tot_pos = npos_before + cpos
        cnt_v[...] = jnp.where(lanes == 0, tot_pos,
                               jnp.where(lanes == 1, B - tot_pos, 0))
        pltpu.sync_copy(cnt_v, cnt_hbm)


def _sc_compact(yt, psc, psr, key, idx, alpha_pad):
    f32 = jnp.float32
    return pl.kernel(
        _sc_compact_body,
        out_type=(jax.ShapeDtypeStruct((2 * B,), f32),
                  jax.ShapeDtypeStruct((2 * B,), f32),
                  jax.ShapeDtypeStruct((2 * B,), jnp.int32),
                  jax.ShapeDtypeStruct((2 * B,), f32),
                  jax.ShapeDtypeStruct((16,), jnp.int32)),
        mesh=plsc.VectorSubcoreMesh(**_MESH),
        scratch_types=[
            pltpu.VMEM((B,), f32),
            pltpu.VMEM((PER_W,), f32),
            pltpu.VMEM((PER_W,), f32),
            pltpu.VMEM((PER_W,), jnp.int32),
            pltpu.VMEM((PER_W,), jnp.int32),
            pltpu.VMEM((PER_W,), f32),
            pltpu.VMEM((PER_W,), jnp.int32),
            pltpu.VMEM((PER_W,), jnp.int32),
            pltpu.VMEM((16,), jnp.int32),
            pltpu.SemaphoreType.DMA,
        ],
        compiler_params=pltpu.CompilerParams(needs_layout_passes=False),
    )(yt, psc, psr, key, idx, alpha_pad)


def _tc_dense_body(cnt, pp_col, ap_col, kp_col, pn_row, kp_row,
                   a_new_ref, win_ref, loss_ref, accS, accT, accL, smacc):
    r = pl.program_id(0)
    num_pos = cnt[0]
    num_neg = cnt[1]

    accS[...] = jnp.zeros((R_BLK, 1), jnp.float32)
    accT[...] = jnp.zeros((R_BLK, 1), jnp.float32)
    accL[...] = jnp.zeros((R_BLK, 1), jnp.float32)

    row_active = r * R_BLK < num_pos
    lanec = lax.broadcasted_iota(jnp.int32, (1, C_BLK), 1)
    pr = pp_col[...]
    ar = ap_col[...]
    cr2 = (LOG_RHO - ar) * LOG2E
    kr = kp_col[...]

    for cc in range(NCB):
        @pl.when(row_active & (cc * C_BLK < num_neg))
        def _main(cc=cc):
            pc = pn_row[:, cc * C_BLK:(cc + 1) * C_BLK]
            negmask = lanec < (num_neg - cc * C_BLK)
            h = jnp.maximum(pc - pr, 0.0)
            e2 = jnp.where(negmask, h * h + cr2, -jnp.inf)
            nabs = lax.bitcast_convert_type(
                lax.bitcast_convert_type(e2, jnp.int32) | jnp.int32(-2147483648),
                jnp.float32)
            u = jnp.exp2(nabs)
            l2 = jnp.log2(1.0 + u)
            m2 = jnp.maximum(e2, 0.0)
            th = jnp.tanh(e2 * (LN2 * 0.5))
            accS[...] += jnp.sum(m2 + l2, axis=1, keepdims=True)
            accT[...] += jnp.sum(th, axis=1, keepdims=True)

        @pl.when(row_active & (cc * C_BLK < num_pos))
        def _dup(cc=cc):
            kc0 = kp_row[:, cc * C_BLK:(cc + 1) * C_BLK]
            posmask = lanec < (num_pos - cc * C_BLK)
            kc = jnp.where(posmask, kc0, jnp.int32(KEY_PAD))
            delta = lax.bitcast_convert_type(kc - kr - 1, jnp.uint32)
            hit = (delta < jnp.uint32(4095)).astype(jnp.float32)
            accL[...] += jnp.sum(hit, axis=1, keepdims=True)

    np_f = num_pos.astype(jnp.float32)
    nn_f = num_neg.astype(jnp.float32)
    nact = (num_neg + (C_BLK - 1)) // C_BLK
    W = 0.5 * ((nact * C_BLK).astype(jnp.float32) + accT[...])
    S = LN2 * accS[...]
    a_new_ref[...] = ar - LR_DUAL * (1.0 - W / nn_f)
    rowi = lax.broadcasted_iota(jnp.int32, (R_BLK, 1), 0) + r * R_BLK
    rowmask = rowi < num_pos
    win_ref[...] = jnp.where(rowmask & (accL[...] == 0.0), 1.0, 0.0)
    part_S = jnp.sum(jnp.where(rowmask, S, 0.0))
    part_a = jnp.sum(jnp.where(rowmask, ar, 0.0))

    @pl.when(r == 0)
    def _init():
        smacc[0] = part_S
        smacc[1] = part_a

    @pl.when(r > 0)
    def _accum():
        smacc[0] = smacc[0] + part_S
        smacc[1] = smacc[1] + part_a

    @pl.when(r == NBR - 1)
    def _loss():
        val = (LAM / RHO) * smacc[0] / (np_f * nn_f) + smacc[1] / np_f
        loss_ref[...] = jnp.reshape(val, (1, 1))


def _tc_dense(cnt, pp_col, ap_col, kp_col, pn_row, kp_row):
    col_spec = pl.BlockSpec((R_BLK, 1), lambda r: (r, 0))
    row_spec = pl.BlockSpec((1, B), lambda r: (0, 0))
    return pl.pallas_call(
        _tc_dense_body,
        grid=(NBR,),
        in_specs=[pl.BlockSpec(memory_space=pltpu.SMEM),
                  col_spec, col_spec, col_spec, row_spec, row_spec],
        out_specs=[pl.BlockSpec((R_BLK, 1), lambda r: (r, 0)),
                   pl.BlockSpec((R_BLK, 1), lambda r: (r, 0)),
                   pl.BlockSpec((1, 1), lambda r: (0, 0))],
        out_shape=[jax.ShapeDtypeStruct((B, 1), jnp.float32),
                   jax.ShapeDtypeStruct((B, 1), jnp.float32),
                   jax.ShapeDtypeStruct((1, 1), jnp.float32)],
        scratch_shapes=[pltpu.VMEM((R_BLK, 1), jnp.float32),
                        pltpu.VMEM((R_BLK, 1), jnp.float32),
                        pltpu.VMEM((R_BLK, 1), jnp.float32),
                        pltpu.SMEM((2,), jnp.float32)],
    )(cnt, pp_col, ap_col, kp_col, pn_row, kp_row)


def _sc_scatter_body(alpha_hbm, key_hbm, val_hbm, win_hbm, out_hbm,
                     tbl_v, key_v, val_v, win_v):
    wid = lax.axis_index("s") * NC + lax.axis_index("c")
    lo = wid * CHUNK
    pltpu.sync_copy(alpha_hbm.at[pl.ds(lo, CHUNK)], tbl_v)
    pltpu.sync_copy(key_hbm, key_v)
    pltpu.sync_copy(val_hbm, val_v)
    pltpu.sync_copy(win_hbm, win_v)

    def body(k, carry):
        off = pl.multiple_of(k * 16, 16)
        iv = key_v[pl.ds(off, 16)] >> 13
        vv = val_v[pl.ds(off, 16)]
        wv = win_v[pl.ds(off, 16)]
        msk = (iv >= lo) & (iv < lo + CHUNK) & (wv > 0.0)
        plsc.store_scatter(tbl_v, [iv - lo], vv, mask=msk)
        return carry

    lax.fori_loop(0, B // 16, body, 0, unroll=8)
    pltpu.sync_copy(tbl_v, out_hbm.at[pl.ds(lo, CHUNK)])


def _sc_scatter(alpha_pad, keys, vals, win):
    return pl.kernel(
        _sc_scatter_body,
        out_type=jax.ShapeDtypeStruct((PAD_LEN,), jnp.float32),
        mesh=plsc.VectorSubcoreMesh(**_MESH),
        scratch_types=[
            pltpu.VMEM((CHUNK,), jnp.float32),
            pltpu.VMEM((B,), jnp.int32),
            pltpu.VMEM((B,), jnp.float32),
            pltpu.VMEM((B,), jnp.float32),
        ],
        compiler_params=pltpu.CompilerParams(needs_layout_passes=False),
    )(alpha_pad, keys, vals, win)


def kernel(y_pred, y_true, index, alpha):
    p = y_pred.reshape(B)
    yt = y_true.reshape(B)
    index = index.reshape(B).astype(jnp.int32)
    alpha_pad = jnp.pad(alpha.reshape(DATA_LEN), (0, PAD_LEN - DATA_LEN))

    slot = jnp.arange(B, dtype=jnp.int32)
    key = index * 8192 + slot
    psc = p * SQRT_LOG2E
    psr = (p + MARGIN) * SQRT_LOG2E

    pp, ap, kp, pn, cnt = _sc_compact(yt, psc, psr, key, index, alpha_pad)

    a_new, win, loss = _tc_dense(
        cnt[:2], pp[:B].reshape(B, 1), ap[:B].reshape(B, 1),
        kp[:B].reshape(B, 1), pn[:B].reshape(1, B), kp[:B].reshape(1, B))

    alpha_out = _sc_scatter(alpha_pad, kp[:B], a_new.reshape(B), win.reshape(B))
    return loss.reshape(()), alpha_out[:DATA_LEN].reshape(DATA_LEN, 1)


# speedup vs baseline: 1.6224x; 1.6224x over previous
import functools
import math

import jax
import jax.numpy as jnp
from jax import lax
from jax.experimental import pallas as pl
from jax.experimental.pallas import tpu as pltpu
from jax.experimental.pallas import tpu_sc as plsc

DATA_LEN = 100000
RHO = 0.001
LR_DUAL = 0.001
MARGIN = 1.0
LAM = 1.0
LOG_RHO = math.log(RHO)

B = 4096
NC, NS = 2, 16
NW = NC * NS
PER_W = B // NW
CHUNK = 3136
PAD_LEN = NW * CHUNK

R_BLK = 256
NBR = B // R_BLK
C_BLK = 512
NCB = B // C_BLK

LOG2E = 1.4426950408889634
LN2 = 0.6931471805599453
SQRT_LOG2E = LOG2E ** 0.5
KEY_PAD = -(2 ** 30)

_MESH = dict(core_axis_name="c", subcore_axis_name="s", num_cores=NC,
             num_subcores=NS)


def _sc_compact_body(yt_hbm, psc_hbm, psr_hbm, key_hbm, idx_hbm, alpha_hbm,
                     ppA_hbm, apA_hbm, kpA_hbm, pnA_hbm,
                     ppB_hbm, apB_hbm, kpB_hbm, pnB_hbm, cnt_hbm,
                     ytf_v, psc_v, psr_v, key_v, idx_v, a_v,
                     tgtp_v, tgtn_v, cnt_v, pp_s, ap_s, kp_s, pn_s, sem):
    cid = lax.axis_index("c")
    sid = lax.axis_index("s")
    wid = cid * NS + sid
    base = wid * PER_W
    d_yt = pltpu.async_copy(yt_hbm, ytf_v, sem)
    d_psc = pltpu.async_copy(psc_hbm.at[pl.ds(base, PER_W)], psc_v, sem)
    d_psr = pltpu.async_copy(psr_hbm.at[pl.ds(base, PER_W)], psr_v, sem)
    d_key = pltpu.async_copy(key_hbm.at[pl.ds(base, PER_W)], key_v, sem)
    d_idx = pltpu.async_copy(idx_hbm.at[pl.ds(base, PER_W)], idx_v, sem)
    d_idx.wait()
    d_ga = pltpu.async_copy(alpha_hbm.at[idx_v], a_v, sem)
    d_yt.wait()

    def pbody(k, s16):
        return s16 + ytf_v[pl.ds(k * 16, 16)]

    acc16 = lax.fori_loop(0, wid * 8, pbody, jnp.zeros((16,), jnp.float32))
    npos_before = jnp.sum(acc16).astype(jnp.int32)
    nneg_before = base - npos_before

    lanes = lax.iota(jnp.int32, 16)
    cpos = jnp.int32(0)
    cneg = jnp.int32(0)
    for v in range(PER_W // 16):
        yv = ytf_v[pl.ds(base + v * 16, 16)]
        posm = yv == 1.0
        pinc = posm.astype(jnp.int32)
        cs = plsc.cumsum(pinc)
        padbase = B + base + v * 16
        tgt_p = jnp.where(posm, npos_before + cpos + cs - 1, padbase + lanes)
        tgt_n = jnp.where(posm, padbase + lanes,
                          nneg_before + cneg + (lanes + 1 - cs) - 1)
        tgtp_v[pl.ds(v * 16, 16)] = tgt_p
        tgtn_v[pl.ds(v * 16, 16)] = tgt_n
        nposv = jnp.sum(pinc)
        cpos = cpos + nposv
        cneg = cneg + (16 - nposv)

    d_psc.wait()
    d_psr.wait()
    d_key.wait()
    d_ga.wait()
    d_pp = pltpu.async_copy(psc_v, pp_s.at[tgtp_v], sem)
    d_ap = pltpu.async_copy(a_v, ap_s.at[tgtp_v], sem)
    d_kp = pltpu.async_copy(key_v, kp_s.at[tgtp_v], sem)
    d_pn = pltpu.async_copy(psr_v, pn_s.at[tgtn_v], sem)
    d_pp.wait()
    d_ap.wait()
    d_kp.wait()
    d_pn.wait()
    plsc.subcore_barrier()

    @pl.when(sid == 0)
    def _flush():
        @pl.when(cid == 0)
        def _a():
            pltpu.sync_copy(pp_s.at[pl.ds(0, B)], ppA_hbm)
            pltpu.sync_copy(ap_s.at[pl.ds(0, B)], apA_hbm)
            pltpu.sync_copy(kp_s.at[pl.ds(0, B)], kpA_hbm)
            pltpu.sync_copy(pn_s.at[pl.ds(0, B)], pnA_hbm)

        @pl.when(cid == 1)
        def _b():
            pltpu.sync_copy(pp_s.at[pl.ds(0, B)], ppB_hbm)
            pltpu.sync_copy(ap_s.at[pl.ds(0, B)], apB_hbm)
            pltpu.sync_copy(kp_s.at[pl.ds(0, B)], kpB_hbm)
            pltpu.sync_copy(pn_s.at[pl.ds(0, B)], pnB_hbm)

    @pl.when(sid == NS - 1)
    def _write_counts():
        tot_pos = npos_before + cpos

        @pl.when(cid == 1)
        def _tot():
            cnt_v[...] = jnp.where(lanes == 0, tot_pos,
                                   jnp.where(lanes == 1, B - tot_pos, 0))
            pltpu.sync_copy(cnt_v.at[pl.ds(0, 8)], cnt_hbm.at[pl.ds(0, 8)])

        @pl.when(cid == 0)
        def _fh():
            cnt_v[...] = jnp.where(lanes == 0, tot_pos,
                                   jnp.where(lanes == 1, B // 2 - tot_pos, 0))
            pltpu.sync_copy(cnt_v.at[pl.ds(0, 8)], cnt_hbm.at[pl.ds(8, 8)])


def _sc_compact(yt, psc, psr, key, idx, alpha_pad):
    f32 = jnp.float32
    arr = lambda dt: jax.ShapeDtypeStruct((B,), dt)
    return pl.kernel(
        _sc_compact_body,
        out_type=(arr(f32), arr(f32), arr(jnp.int32), arr(f32),
                  arr(f32), arr(f32), arr(jnp.int32), arr(f32),
                  jax.ShapeDtypeStruct((16,), jnp.int32)),
        mesh=plsc.VectorSubcoreMesh(**_MESH),
        scratch_types=[
            pltpu.VMEM((B,), f32),
            pltpu.VMEM((PER_W,), f32),
            pltpu.VMEM((PER_W,), f32),
            pltpu.VMEM((PER_W,), jnp.int32),
            pltpu.VMEM((PER_W,), jnp.int32),
            pltpu.VMEM((PER_W,), f32),
            pltpu.VMEM((PER_W,), jnp.int32),
            pltpu.VMEM((PER_W,), jnp.int32),
            pltpu.VMEM((16,), jnp.int32),
            pltpu.VMEM_SHARED((2 * B,), f32),
            pltpu.VMEM_SHARED((2 * B,), f32),
            pltpu.VMEM_SHARED((2 * B,), jnp.int32),
            pltpu.VMEM_SHARED((2 * B,), f32),
            pltpu.SemaphoreType.DMA,
        ],
        compiler_params=pltpu.CompilerParams(needs_layout_passes=False),
    )(yt, psc, psr, key, idx, alpha_pad)


def _tc_dense_body(cnt, ppA_col, apA_col, kpA_col, ppB_col, apB_col, kpB_col,
                   pnA_row, kpA_row, pnB_row, kpB_row,
                   a_new_ref, win_ref, kout_ref, loss_ref,
                   accS, accT, accL, smacc):
    r = pl.program_id(0)
    num_pos = cnt[0]
    num_neg = cnt[1]
    npfh = cnt[8]
    nnfh = cnt[9]

    accS[...] = jnp.zeros((R_BLK, 1), jnp.float32)
    accT[...] = jnp.zeros((R_BLK, 1), jnp.float32)
    accL[...] = jnp.zeros((R_BLK, 1), jnp.float32)

    row_active = r * R_BLK < num_pos
    lanec = lax.broadcasted_iota(jnp.int32, (1, C_BLK), 1)
    rowi = lax.broadcasted_iota(jnp.int32, (R_BLK, 1), 0) + r * R_BLK
    rowselA = rowi < npfh
    pr = jnp.where(rowselA, ppA_col[...], ppB_col[...])
    ar = jnp.where(rowselA, apA_col[...], apB_col[...])
    kr = jnp.where(rowselA, kpA_col[...], kpB_col[...])
    cr2 = (LOG_RHO - ar) * LOG2E
    kout_ref[...] = kr

    for cc in range(NCB):
        lanec_g = lanec + cc * C_BLK

        @pl.when(row_active & (cc * C_BLK < num_neg))
        def _main(cc=cc, lanec_g=lanec_g):
            pc = jnp.where(lanec_g < nnfh,
                           pnA_row[:, cc * C_BLK:(cc + 1) * C_BLK],
                           pnB_row[:, cc * C_BLK:(cc + 1) * C_BLK])
            negmask = lanec_g < num_neg
            h = jnp.maximum(pc - pr, 0.0)
            e2 = jnp.where(negmask, h * h + cr2, -jnp.inf)
            nabs = lax.bitcast_convert_type(
                lax.bitcast_convert_type(e2, jnp.int32) | jnp.int32(-2147483648),
                jnp.float32)
            u = jnp.exp2(nabs)
            l2 = jnp.log2(1.0 + u)
            m2 = jnp.maximum(e2, 0.0)
            th = jnp.tanh(e2 * (LN2 * 0.5))
            accS[...] += jnp.sum(m2 + l2, axis=1, keepdims=True)
            accT[...] += jnp.sum(th, axis=1, keepdims=True)

        @pl.when(row_active & (cc * C_BLK < num_pos))
        def _dup(cc=cc, lanec_g=lanec_g):
            kc0 = jnp.where(lanec_g < npfh,
                            kpA_row[:, cc * C_BLK:(cc + 1) * C_BLK],
                            kpB_row[:, cc * C_BLK:(cc + 1) * C_BLK])
            kc = jnp.where(lanec_g < num_pos, kc0, jnp.int32(KEY_PAD))
            delta = lax.bitcast_convert_type(kc - kr - 1, jnp.uint32)
            hit = (delta < jnp.uint32(4095)).astype(jnp.float32)
            accL[...] += jnp.sum(hit, axis=1, keepdims=True)

    np_f = num_pos.astype(jnp.float32)
    nn_f = num_neg.astype(jnp.float32)
    nact = (num_neg + (C_BLK - 1)) // C_BLK
    W = 0.5 * ((nact * C_BLK).astype(jnp.float32) + accT[...])
    S = LN2 * accS[...]
    a_new_ref[...] = ar - LR_DUAL * (1.0 - W / nn_f)
    rowmask = rowi < num_pos
    win_ref[...] = jnp.where(rowmask & (accL[...] == 0.0), 1.0, 0.0)
    part_S = jnp.sum(jnp.where(rowmask, S, 0.0))
    part_a = jnp.sum(jnp.where(rowmask, ar, 0.0))

    @pl.when(r == 0)
    def _init():
        smacc[0] = part_S
        smacc[1] = part_a

    @pl.when(r > 0)
    def _accum():
        smacc[0] = smacc[0] + part_S
        smacc[1] = smacc[1] + part_a

    @pl.when(r == NBR - 1)
    def _loss():
        val = (LAM / RHO) * smacc[0] / (np_f * nn_f) + smacc[1] / np_f
        loss_ref[...] = jnp.reshape(val, (1, 1))


def _tc_dense(cnt, ppA, apA, kpA, ppB, apB, kpB, pnA, pnB):
    col_spec = pl.BlockSpec((R_BLK, 1), lambda r: (r, 0))
    row_spec = pl.BlockSpec((1, B), lambda r: (0, 0))
    c2 = lambda x: x.reshape(B, 1)
    r2 = lambda x: x.reshape(1, B)
    return pl.pallas_call(
        _tc_dense_body,
        grid=(NBR,),
        in_specs=[pl.BlockSpec(memory_space=pltpu.SMEM),
                  col_spec, col_spec, col_spec, col_spec, col_spec, col_spec,
                  row_spec, row_spec, row_spec, row_spec],
        out_specs=[pl.BlockSpec((R_BLK, 1), lambda r: (r, 0)),
                   pl.BlockSpec((R_BLK, 1), lambda r: (r, 0)),
                   pl.BlockSpec((R_BLK, 1), lambda r: (r, 0)),
                   pl.BlockSpec((1, 1), lambda r: (0, 0))],
        out_shape=[jax.ShapeDtypeStruct((B, 1), jnp.float32),
                   jax.ShapeDtypeStruct((B, 1), jnp.float32),
                   jax.ShapeDtypeStruct((B, 1), jnp.int32),
                   jax.ShapeDtypeStruct((1, 1), jnp.float32)],
        scratch_shapes=[pltpu.VMEM((R_BLK, 1), jnp.float32),
                        pltpu.VMEM((R_BLK, 1), jnp.float32),
                        pltpu.VMEM((R_BLK, 1), jnp.float32),
                        pltpu.SMEM((2,), jnp.float32)],
    )(cnt, c2(ppA), c2(apA), c2(kpA), c2(ppB), c2(apB), c2(kpB),
      r2(pnA), r2(kpA), r2(pnB), r2(kpB))


def _sc_scatter_body(alpha_hbm, key_hbm, val_hbm, win_hbm, out_hbm,
                     tbl_v, key_v, val_v, win_v):
    wid = lax.axis_index("s") * NC + lax.axis_index("c")
    lo = wid * CHUNK
    pltpu.sync_copy(alpha_hbm.at[pl.ds(lo, CHUNK)], tbl_v)
    pltpu.sync_copy(key_hbm, key_v)
    pltpu.sync_copy(val_hbm, val_v)
    pltpu.sync_copy(win_hbm, win_v)

    def body(k, carry):
        off = pl.multiple_of(k * 16, 16)
        iv = key_v[pl.ds(off, 16)] >> 13
        vv = val_v[pl.ds(off, 16)]
        wv = win_v[pl.ds(off, 16)]
        msk = (iv >= lo) & (iv < lo + CHUNK) & (wv > 0.0)
        plsc.store_scatter(tbl_v, [iv - lo], vv, mask=msk)
        return carry

    lax.fori_loop(0, B // 16, body, 0, unroll=8)
    pltpu.sync_copy(tbl_v, out_hbm.at[pl.ds(lo, CHUNK)])


def _sc_scatter(alpha_pad, keys, vals, win):
    return pl.kernel(
        _sc_scatter_body,
        out_type=jax.ShapeDtypeStruct((PAD_LEN,), jnp.float32),
        mesh=plsc.VectorSubcoreMesh(**_MESH),
        scratch_types=[
            pltpu.VMEM((CHUNK,), jnp.float32),
            pltpu.VMEM((B,), jnp.int32),
            pltpu.VMEM((B,), jnp.float32),
            pltpu.VMEM((B,), jnp.float32),
        ],
        compiler_params=pltpu.CompilerParams(needs_layout_passes=False),
    )(alpha_pad, keys, vals, win)


def kernel(y_pred, y_true, index, alpha):
    p = y_pred.reshape(B)
    yt = y_true.reshape(B)
    index = index.reshape(B).astype(jnp.int32)
    alpha_pad = jnp.pad(alpha.reshape(DATA_LEN), (0, PAD_LEN - DATA_LEN))

    slot = jnp.arange(B, dtype=jnp.int32)
    key = index * 8192 + slot
    psc = p * SQRT_LOG2E
    psr = (p + MARGIN) * SQRT_LOG2E

    (ppA, apA, kpA, pnA, ppB, apB, kpB, pnB, cnt) = _sc_compact(
        yt, psc, psr, key, index, alpha_pad)

    a_new, win, kout, loss = _tc_dense(cnt, ppA, apA, kpA, ppB, apB, kpB,
                                       pnA, pnB)

    alpha_out = _sc_scatter(alpha_pad, kout.reshape(B), a_new.reshape(B),
                            win.reshape(B))
    return loss.reshape(()), alpha_out[:DATA_LEN].reshape(DATA_LEN, 1)


# speedup vs baseline: 1.8095x; 1.1153x over previous
import functools
import math

import jax
import jax.numpy as jnp
from jax import lax
from jax.experimental import pallas as pl
from jax.experimental.pallas import tpu as pltpu
from jax.experimental.pallas import tpu_sc as plsc

DATA_LEN = 100000
RHO = 0.001
LR_DUAL = 0.001
MARGIN = 1.0
LAM = 1.0
LOG_RHO = math.log(RHO)

B = 4096
NC, NS = 2, 16
NW = NC * NS
PER_W = B // NW
CHUNK = 3136
PAD_LEN = NW * CHUNK

R_BLK = 512
NBR = B // R_BLK
C_BLK = 512
NCB = B // C_BLK

LOG2E = 1.4426950408889634
LN2 = 0.6931471805599453
SQRT_LOG2E = LOG2E ** 0.5
KEY_PAD = -(2 ** 30)

_MESH = dict(core_axis_name="c", subcore_axis_name="s", num_cores=NC,
             num_subcores=NS)


def _sc_compact_body(yt_hbm, psc_hbm, psr_hbm, key_hbm, idx_hbm, alpha_hbm,
                     ppA_hbm, apA_hbm, kpA_hbm, pnA_hbm,
                     ppB_hbm, apB_hbm, kpB_hbm, pnB_hbm, cnt_hbm,
                     ytf_v, psc_v, psr_v, key_v, idx_v, a_v,
                     tgtp_v, tgtn_v, cnt_v, pp_s, ap_s, kp_s, pn_s, sem):
    cid = lax.axis_index("c")
    sid = lax.axis_index("s")
    wid = cid * NS + sid
    base = wid * PER_W
    d_yt = pltpu.async_copy(yt_hbm, ytf_v, sem)
    d_psc = pltpu.async_copy(psc_hbm.at[pl.ds(base, PER_W)], psc_v, sem)
    d_psr = pltpu.async_copy(psr_hbm.at[pl.ds(base, PER_W)], psr_v, sem)
    d_key = pltpu.async_copy(key_hbm.at[pl.ds(base, PER_W)], key_v, sem)
    d_idx = pltpu.async_copy(idx_hbm.at[pl.ds(base, PER_W)], idx_v, sem)
    d_idx.wait()
    d_ga = pltpu.async_copy(alpha_hbm.at[idx_v], a_v, sem)
    d_yt.wait()

    def pbody(k, s16):
        return s16 + ytf_v[pl.ds(k * 16, 16)]

    acc16 = lax.fori_loop(0, wid * 8, pbody, jnp.zeros((16,), jnp.float32))
    npos_before = jnp.sum(acc16).astype(jnp.int32)
    nneg_before = base - npos_before

    lanes = lax.iota(jnp.int32, 16)
    cpos = jnp.int32(0)
    cneg = jnp.int32(0)
    for v in range(PER_W // 16):
        yv = ytf_v[pl.ds(base + v * 16, 16)]
        posm = yv == 1.0
        pinc = posm.astype(jnp.int32)
        cs = plsc.cumsum(pinc)
        padbase = B + base + v * 16
        tgt_p = jnp.where(posm, npos_before + cpos + cs - 1, padbase + lanes)
        tgt_n = jnp.where(posm, padbase + lanes,
                          nneg_before + cneg + (lanes + 1 - cs) - 1)
        tgtp_v[pl.ds(v * 16, 16)] = tgt_p
        tgtn_v[pl.ds(v * 16, 16)] = tgt_n
        nposv = jnp.sum(pinc)
        cpos = cpos + nposv
        cneg = cneg + (16 - nposv)

    d_psc.wait()
    d_psr.wait()
    d_key.wait()
    d_ga.wait()
    d_pp = pltpu.async_copy(psc_v, pp_s.at[tgtp_v], sem)
    d_ap = pltpu.async_copy(a_v, ap_s.at[tgtp_v], sem)
    d_kp = pltpu.async_copy(key_v, kp_s.at[tgtp_v], sem)
    d_pn = pltpu.async_copy(psr_v, pn_s.at[tgtn_v], sem)
    d_pp.wait()
    d_ap.wait()
    d_kp.wait()
    d_pn.wait()
    plsc.subcore_barrier()

    @pl.when(sid == 0)
    def _flush():
        @pl.when(cid == 0)
        def _a():
            pltpu.sync_copy(pp_s.at[pl.ds(0, B)], ppA_hbm)
            pltpu.sync_copy(ap_s.at[pl.ds(0, B)], apA_hbm)
            pltpu.sync_copy(kp_s.at[pl.ds(0, B)], kpA_hbm)
            pltpu.sync_copy(pn_s.at[pl.ds(0, B)], pnA_hbm)

        @pl.when(cid == 1)
        def _b():
            pltpu.sync_copy(pp_s.at[pl.ds(0, B)], ppB_hbm)
            pltpu.sync_copy(ap_s.at[pl.ds(0, B)], apB_hbm)
            pltpu.sync_copy(kp_s.at[pl.ds(0, B)], kpB_hbm)
            pltpu.sync_copy(pn_s.at[pl.ds(0, B)], pnB_hbm)

    @pl.when(sid == NS - 1)
    def _write_counts():
        tot_pos = npos_before + cpos

        @pl.when(cid == 1)
        def _tot():
            cnt_v[...] = jnp.where(lanes == 0, tot_pos,
                                   jnp.where(lanes == 1, B - tot_pos, 0))
            pltpu.sync_copy(cnt_v.at[pl.ds(0, 8)], cnt_hbm.at[pl.ds(0, 8)])

        @pl.when(cid == 0)
        def _fh():
            cnt_v[...] = jnp.where(lanes == 0, tot_pos,
                                   jnp.where(lanes == 1, B // 2 - tot_pos, 0))
            pltpu.sync_copy(cnt_v.at[pl.ds(0, 8)], cnt_hbm.at[pl.ds(8, 8)])


def _sc_compact(yt, psc, psr, key, idx, alpha_pad):
    f32 = jnp.float32
    arr = lambda dt: jax.ShapeDtypeStruct((B,), dt)
    return pl.kernel(
        _sc_compact_body,
        out_type=(arr(f32), arr(f32), arr(jnp.int32), arr(f32),
                  arr(f32), arr(f32), arr(jnp.int32), arr(f32),
                  jax.ShapeDtypeStruct((16,), jnp.int32)),
        mesh=plsc.VectorSubcoreMesh(**_MESH),
        scratch_types=[
            pltpu.VMEM((B,), f32),
            pltpu.VMEM((PER_W,), f32),
            pltpu.VMEM((PER_W,), f32),
            pltpu.VMEM((PER_W,), jnp.int32),
            pltpu.VMEM((PER_W,), jnp.int32),
            pltpu.VMEM((PER_W,), f32),
            pltpu.VMEM((PER_W,), jnp.int32),
            pltpu.VMEM((PER_W,), jnp.int32),
            pltpu.VMEM((16,), jnp.int32),
            pltpu.VMEM_SHARED((2 * B,), f32),
            pltpu.VMEM_SHARED((2 * B,), f32),
            pltpu.VMEM_SHARED((2 * B,), jnp.int32),
            pltpu.VMEM_SHARED((2 * B,), f32),
            pltpu.SemaphoreType.DMA,
        ],
        compiler_params=pltpu.CompilerParams(needs_layout_passes=False),
    )(yt, psc, psr, key, idx, alpha_pad)


def _tc_dense_body(cnt, ppA_col, apA_col, kpA_col, ppB_col, apB_col, kpB_col,
                   pnA_row, kpA_row, pnB_row, kpB_row,
                   a_new_ref, win_ref, kout_ref, loss_ref,
                   accS, accT, accL, smacc):
    r = pl.program_id(0)
    num_pos = cnt[0]
    num_neg = cnt[1]
    npfh = cnt[8]
    nnfh = cnt[9]

    accS[...] = jnp.zeros((R_BLK, 128), jnp.float32)
    accT[...] = jnp.zeros((R_BLK, 128), jnp.float32)
    accL[...] = jnp.zeros((R_BLK, 128), jnp.float32)

    def lanefold(x):
        out = x[:, 0:128]
        for q in range(1, C_BLK // 128):
            out = out + x[:, q * 128:(q + 1) * 128]
        return out

    row_active = r * R_BLK < num_pos
    lanec = lax.broadcasted_iota(jnp.int32, (1, C_BLK), 1)
    rowi = lax.broadcasted_iota(jnp.int32, (R_BLK, 1), 0) + r * R_BLK
    rowselA = rowi < npfh
    pr = jnp.where(rowselA, ppA_col[...], ppB_col[...])
    ar = jnp.where(rowselA, apA_col[...], apB_col[...])
    kr = jnp.where(rowselA, kpA_col[...], kpB_col[...])
    cr2 = (LOG_RHO - ar) * LOG2E
    kout_ref[...] = kr

    for cc in range(NCB):
        lanec_g = lanec + cc * C_BLK

        @pl.when(row_active & (cc * C_BLK < num_neg))
        def _main(cc=cc, lanec_g=lanec_g):
            pc = jnp.where(lanec_g < nnfh,
                           pnA_row[:, cc * C_BLK:(cc + 1) * C_BLK],
                           pnB_row[:, cc * C_BLK:(cc + 1) * C_BLK])
            negmask = lanec_g < num_neg
            h = jnp.maximum(pc - pr, 0.0)
            e2 = jnp.where(negmask, h * h + cr2, -jnp.inf)
            nabs = lax.bitcast_convert_type(
                lax.bitcast_convert_type(e2, jnp.int32) | jnp.int32(-2147483648),
                jnp.float32)
            u = jnp.exp2(nabs)
            l2 = jnp.log2(1.0 + u)
            m2 = jnp.maximum(e2, 0.0)
            th = jnp.tanh(e2 * (LN2 * 0.5))
            accS[...] += lanefold(m2 + l2)
            accT[...] += lanefold(th)

        @pl.when(row_active & (cc * C_BLK < num_pos))
        def _dup(cc=cc, lanec_g=lanec_g):
            kc0 = jnp.where(lanec_g < npfh,
                            kpA_row[:, cc * C_BLK:(cc + 1) * C_BLK],
                            kpB_row[:, cc * C_BLK:(cc + 1) * C_BLK])
            kc = jnp.where(lanec_g < num_pos, kc0, jnp.int32(KEY_PAD))
            delta = lax.bitcast_convert_type(kc - kr - 1, jnp.uint32)
            hit = (delta < jnp.uint32(4095)).astype(jnp.float32)
            accL[...] += lanefold(hit)

    np_f = num_pos.astype(jnp.float32)
    nn_f = num_neg.astype(jnp.float32)
    nact = (num_neg + (C_BLK - 1)) // C_BLK
    W = 0.5 * ((nact * C_BLK).astype(jnp.float32)
               + jnp.sum(accT[...], axis=1, keepdims=True))
    S = LN2 * jnp.sum(accS[...], axis=1, keepdims=True)
    a_new_ref[...] = ar - LR_DUAL * (1.0 - W / nn_f)
    rowmask = rowi < num_pos
    lose = jnp.sum(accL[...], axis=1, keepdims=True) > 0.0
    win_ref[...] = jnp.where(rowmask & (~lose), 1.0, 0.0)
    part_S = jnp.sum(jnp.where(rowmask, S, 0.0))
    part_a = jnp.sum(jnp.where(rowmask, ar, 0.0))

    @pl.when(r == 0)
    def _init():
        smacc[0] = part_S
        smacc[1] = part_a

    @pl.when(r > 0)
    def _accum():
        smacc[0] = smacc[0] + part_S
        smacc[1] = smacc[1] + part_a

    @pl.when(r == NBR - 1)
    def _loss():
        val = (LAM / RHO) * smacc[0] / (np_f * nn_f) + smacc[1] / np_f
        loss_ref[...] = jnp.reshape(val, (1, 1))


def _tc_dense(cnt, ppA, apA, kpA, ppB, apB, kpB, pnA, pnB):
    col_spec = pl.BlockSpec((R_BLK, 1), lambda r: (r, 0))
    row_spec = pl.BlockSpec((1, B), lambda r: (0, 0))
    c2 = lambda x: x.reshape(B, 1)
    r2 = lambda x: x.reshape(1, B)
    return pl.pallas_call(
        _tc_dense_body,
        grid=(NBR,),
        in_specs=[pl.BlockSpec(memory_space=pltpu.SMEM),
                  col_spec, col_spec, col_spec, col_spec, col_spec, col_spec,
                  row_spec, row_spec, row_spec, row_spec],
        out_specs=[pl.BlockSpec((R_BLK, 1), lambda r: (r, 0)),
                   pl.BlockSpec((R_BLK, 1), lambda r: (r, 0)),
                   pl.BlockSpec((R_BLK, 1), lambda r: (r, 0)),
                   pl.BlockSpec((1, 1), lambda r: (0, 0))],
        out_shape=[jax.ShapeDtypeStruct((B, 1), jnp.float32),
                   jax.ShapeDtypeStruct((B, 1), jnp.float32),
                   jax.ShapeDtypeStruct((B, 1), jnp.int32),
                   jax.ShapeDtypeStruct((1, 1), jnp.float32)],
        scratch_shapes=[pltpu.VMEM((R_BLK, 128), jnp.float32),
                        pltpu.VMEM((R_BLK, 128), jnp.float32),
                        pltpu.VMEM((R_BLK, 128), jnp.float32),
                        pltpu.SMEM((2,), jnp.float32)],
    )(cnt, c2(ppA), c2(apA), c2(kpA), c2(ppB), c2(apB), c2(kpB),
      r2(pnA), r2(kpA), r2(pnB), r2(kpB))


def _sc_scatter_body(alpha_hbm, key_hbm, val_hbm, win_hbm, out_hbm,
                     tbl_v, key_v, val_v, win_v):
    wid = lax.axis_index("s") * NC + lax.axis_index("c")
    lo = wid * CHUNK
    pltpu.sync_copy(alpha_hbm.at[pl.ds(lo, CHUNK)], tbl_v)
    pltpu.sync_copy(key_hbm, key_v)
    pltpu.sync_copy(val_hbm, val_v)
    pltpu.sync_copy(win_hbm, win_v)

    def body(k, carry):
        off = pl.multiple_of(k * 16, 16)
        iv = key_v[pl.ds(off, 16)] >> 13
        vv = val_v[pl.ds(off, 16)]
        wv = win_v[pl.ds(off, 16)]
        msk = (iv >= lo) & (iv < lo + CHUNK) & (wv > 0.0)
        plsc.store_scatter(tbl_v, [iv - lo], vv, mask=msk)
        return carry

    lax.fori_loop(0, B // 16, body, 0, unroll=8)
    pltpu.sync_copy(tbl_v, out_hbm.at[pl.ds(lo, CHUNK)])


def _sc_scatter(alpha_pad, keys, vals, win):
    return pl.kernel(
        _sc_scatter_body,
        out_type=jax.ShapeDtypeStruct((PAD_LEN,), jnp.float32),
        mesh=plsc.VectorSubcoreMesh(**_MESH),
        scratch_types=[
            pltpu.VMEM((CHUNK,), jnp.float32),
            pltpu.VMEM((B,), jnp.int32),
            pltpu.VMEM((B,), jnp.float32),
            pltpu.VMEM((B,), jnp.float32),
        ],
        compiler_params=pltpu.CompilerParams(needs_layout_passes=False),
    )(alpha_pad, keys, vals, win)


def kernel(y_pred, y_true, index, alpha):
    p = y_pred.reshape(B)
    yt = y_true.reshape(B)
    index = index.reshape(B).astype(jnp.int32)
    alpha_pad = jnp.pad(alpha.reshape(DATA_LEN), (0, PAD_LEN - DATA_LEN))

    slot = jnp.arange(B, dtype=jnp.int32)
    key = index * 8192 + slot
    psc = p * SQRT_LOG2E
    psr = (p + MARGIN) * SQRT_LOG2E

    (ppA, apA, kpA, pnA, ppB, apB, kpB, pnB, cnt) = _sc_compact(
        yt, psc, psr, key, index, alpha_pad)

    a_new, win, kout, loss = _tc_dense(cnt, ppA, apA, kpA, ppB, apB, kpB,
                                       pnA, pnB)

    alpha_out = _sc_scatter(alpha_pad, kout.reshape(B), a_new.reshape(B),
                            win.reshape(B))
    return loss.reshape(()), alpha_out[:DATA_LEN].reshape(DATA_LEN, 1)


# speedup vs baseline: 1.8307x; 1.0117x over previous
import functools
import math

import jax
import jax.numpy as jnp
from jax import lax
from jax.experimental import pallas as pl
from jax.experimental.pallas import tpu as pltpu
from jax.experimental.pallas import tpu_sc as plsc

DATA_LEN = 100000
RHO = 0.001
LR_DUAL = 0.001
MARGIN = 1.0
LAM = 1.0
LOG_RHO = math.log(RHO)

B = 4096
NC, NS = 2, 16
NW = NC * NS
PER_W = B // NW
CHUNK = 3136
PAD_LEN = NW * CHUNK

R_BLK = 512
NBR = B // R_BLK
C_BLK = 1024
NCB = B // C_BLK

LOG2E = 1.4426950408889634
LN2 = 0.6931471805599453
SQRT_LOG2E = LOG2E ** 0.5
KEY_PAD = -(2 ** 30)

_MESH = dict(core_axis_name="c", subcore_axis_name="s", num_cores=NC,
             num_subcores=NS)


def _sc_compact_body(yt_hbm, psc_hbm, psr_hbm, key_hbm, idx_hbm, alpha_hbm,
                     ppA_hbm, apA_hbm, kpA_hbm, pnA_hbm,
                     ppB_hbm, apB_hbm, kpB_hbm, pnB_hbm, cnt_hbm,
                     ytf_v, psc_v, psr_v, key_v, idx_v, a_v,
                     tgtp_v, tgtn_v, cnt_v, pp_s, ap_s, kp_s, pn_s, sem):
    cid = lax.axis_index("c")
    sid = lax.axis_index("s")
    wid = cid * NS + sid
    base = wid * PER_W
    d_yt = pltpu.async_copy(yt_hbm, ytf_v, sem)
    d_psc = pltpu.async_copy(psc_hbm.at[pl.ds(base, PER_W)], psc_v, sem)
    d_psr = pltpu.async_copy(psr_hbm.at[pl.ds(base, PER_W)], psr_v, sem)
    d_key = pltpu.async_copy(key_hbm.at[pl.ds(base, PER_W)], key_v, sem)
    d_idx = pltpu.async_copy(idx_hbm.at[pl.ds(base, PER_W)], idx_v, sem)
    d_idx.wait()
    d_ga = pltpu.async_copy(alpha_hbm.at[idx_v], a_v, sem)
    d_yt.wait()

    def pbody(k, s16):
        return s16 + ytf_v[pl.ds(k * 16, 16)]

    acc16 = lax.fori_loop(0, wid * 8, pbody, jnp.zeros((16,), jnp.float32))
    npos_before = jnp.sum(acc16).astype(jnp.int32)
    nneg_before = base - npos_before

    lanes = lax.iota(jnp.int32, 16)
    cpos = jnp.int32(0)
    cneg = jnp.int32(0)
    for v in range(PER_W // 16):
        yv = ytf_v[pl.ds(base + v * 16, 16)]
        posm = yv == 1.0
        pinc = posm.astype(jnp.int32)
        cs = plsc.cumsum(pinc)
        padbase = B + base + v * 16
        tgt_p = jnp.where(posm, npos_before + cpos + cs - 1, padbase + lanes)
        tgt_n = jnp.where(posm, padbase + lanes,
                          nneg_before + cneg + (lanes + 1 - cs) - 1)
        tgtp_v[pl.ds(v * 16, 16)] = tgt_p
        tgtn_v[pl.ds(v * 16, 16)] = tgt_n
        nposv = jnp.sum(pinc)
        cpos = cpos + nposv
        cneg = cneg + (16 - nposv)

    d_psc.wait()
    d_psr.wait()
    d_key.wait()
    d_ga.wait()
    d_pp = pltpu.async_copy(psc_v, pp_s.at[tgtp_v], sem)
    d_ap = pltpu.async_copy(a_v, ap_s.at[tgtp_v], sem)
    d_kp = pltpu.async_copy(key_v, kp_s.at[tgtp_v], sem)
    d_pn = pltpu.async_copy(psr_v, pn_s.at[tgtn_v], sem)
    d_pp.wait()
    d_ap.wait()
    d_kp.wait()
    d_pn.wait()
    plsc.subcore_barrier()

    @pl.when(sid == 0)
    def _flush():
        @pl.when(cid == 0)
        def _a():
            pltpu.sync_copy(pp_s.at[pl.ds(0, B)], ppA_hbm)
            pltpu.sync_copy(ap_s.at[pl.ds(0, B)], apA_hbm)
            pltpu.sync_copy(kp_s.at[pl.ds(0, B)], kpA_hbm)
            pltpu.sync_copy(pn_s.at[pl.ds(0, B)], pnA_hbm)

        @pl.when(cid == 1)
        def _b():
            pltpu.sync_copy(pp_s.at[pl.ds(0, B)], ppB_hbm)
            pltpu.sync_copy(ap_s.at[pl.ds(0, B)], apB_hbm)
            pltpu.sync_copy(kp_s.at[pl.ds(0, B)], kpB_hbm)
            pltpu.sync_copy(pn_s.at[pl.ds(0, B)], pnB_hbm)

    @pl.when(sid == NS - 1)
    def _write_counts():
        tot_pos = npos_before + cpos

        @pl.when(cid == 1)
        def _tot():
            cnt_v[...] = jnp.where(lanes == 0, tot_pos,
                                   jnp.where(lanes == 1, B - tot_pos, 0))
            pltpu.sync_copy(cnt_v.at[pl.ds(0, 8)], cnt_hbm.at[pl.ds(0, 8)])

        @pl.when(cid == 0)
        def _fh():
            cnt_v[...] = jnp.where(lanes == 0, tot_pos,
                                   jnp.where(lanes == 1, B // 2 - tot_pos, 0))
            pltpu.sync_copy(cnt_v.at[pl.ds(0, 8)], cnt_hbm.at[pl.ds(8, 8)])


def _sc_compact(yt, psc, psr, key, idx, alpha_pad):
    f32 = jnp.float32
    arr = lambda dt: jax.ShapeDtypeStruct((B,), dt)
    return pl.kernel(
        _sc_compact_body,
        out_type=(arr(f32), arr(f32), arr(jnp.int32), arr(f32),
                  arr(f32), arr(f32), arr(jnp.int32), arr(f32),
                  jax.ShapeDtypeStruct((16,), jnp.int32)),
        mesh=plsc.VectorSubcoreMesh(**_MESH),
        scratch_types=[
            pltpu.VMEM((B,), f32),
            pltpu.VMEM((PER_W,), f32),
            pltpu.VMEM((PER_W,), f32),
            pltpu.VMEM((PER_W,), jnp.int32),
            pltpu.VMEM((PER_W,), jnp.int32),
            pltpu.VMEM((PER_W,), f32),
            pltpu.VMEM((PER_W,), jnp.int32),
            pltpu.VMEM((PER_W,), jnp.int32),
            pltpu.VMEM((16,), jnp.int32),
            pltpu.VMEM_SHARED((2 * B,), f32),
            pltpu.VMEM_SHARED((2 * B,), f32),
            pltpu.VMEM_SHARED((2 * B,), jnp.int32),
            pltpu.VMEM_SHARED((2 * B,), f32),
            pltpu.SemaphoreType.DMA,
        ],
        compiler_params=pltpu.CompilerParams(needs_layout_passes=False),
    )(yt, psc, psr, key, idx, alpha_pad)


def _tc_dense_body(cnt, ppA_col, apA_col, kpA_col, ppB_col, apB_col, kpB_col,
                   pnA_row, kpA_row, pnB_row, kpB_row,
                   a_new_ref, win_ref, kout_ref, loss_ref,
                   accS, accT, accL, smacc):
    r = pl.program_id(0)
    num_pos = cnt[0]
    num_neg = cnt[1]
    npfh = cnt[8]
    nnfh = cnt[9]

    accS[...] = jnp.zeros((R_BLK, 128), jnp.float32)
    accT[...] = jnp.zeros((R_BLK, 128), jnp.float32)
    accL[...] = jnp.zeros((R_BLK, 128), jnp.float32)

    def lanefold(x):
        out = x[:, 0:128]
        for q in range(1, C_BLK // 128):
            out = out + x[:, q * 128:(q + 1) * 128]
        return out

    row_active = r * R_BLK < num_pos
    lanec = lax.broadcasted_iota(jnp.int32, (1, C_BLK), 1)
    rowi = lax.broadcasted_iota(jnp.int32, (R_BLK, 1), 0) + r * R_BLK
    rowselA = rowi < npfh
    pr = jnp.where(rowselA, ppA_col[...], ppB_col[...])
    ar = jnp.where(rowselA, apA_col[...], apB_col[...])
    kr = jnp.where(rowselA, kpA_col[...], kpB_col[...])
    cr2 = (LOG_RHO - ar) * LOG2E
    kout_ref[...] = kr

    for cc in range(NCB):
        lanec_g = lanec + cc * C_BLK

        @pl.when(row_active & (cc * C_BLK < num_neg))
        def _main(cc=cc, lanec_g=lanec_g):
            pc = jnp.where(lanec_g < nnfh,
                           pnA_row[:, cc * C_BLK:(cc + 1) * C_BLK],
                           pnB_row[:, cc * C_BLK:(cc + 1) * C_BLK])
            negmask = lanec_g < num_neg
            h = jnp.maximum(pc - pr, 0.0)
            e2 = jnp.where(negmask, h * h + cr2, -jnp.inf)
            nabs = lax.bitcast_convert_type(
                lax.bitcast_convert_type(e2, jnp.int32) | jnp.int32(-2147483648),
                jnp.float32)
            u = jnp.exp2(nabs)
            l2 = jnp.log2(1.0 + u)
            m2 = jnp.maximum(e2, 0.0)
            th = jnp.tanh(e2 * (LN2 * 0.5))
            accS[...] += lanefold(m2 + l2)
            accT[...] += lanefold(th)

        @pl.when(row_active & (cc * C_BLK < num_pos))
        def _dup(cc=cc, lanec_g=lanec_g):
            kc0 = jnp.where(lanec_g < npfh,
                            kpA_row[:, cc * C_BLK:(cc + 1) * C_BLK],
                            kpB_row[:, cc * C_BLK:(cc + 1) * C_BLK])
            kc = jnp.where(lanec_g < num_pos, kc0, jnp.int32(KEY_PAD))
            delta = lax.bitcast_convert_type(kc - kr - 1, jnp.uint32)
            hit = (delta < jnp.uint32(4095)).astype(jnp.float32)
            accL[...] += lanefold(hit)

    np_f = num_pos.astype(jnp.float32)
    nn_f = num_neg.astype(jnp.float32)
    nact = (num_neg + (C_BLK - 1)) // C_BLK
    W = 0.5 * ((nact * C_BLK).astype(jnp.float32)
               + jnp.sum(accT[...], axis=1, keepdims=True))
    S = LN2 * jnp.sum(accS[...], axis=1, keepdims=True)
    a_new_ref[...] = ar - LR_DUAL * (1.0 - W / nn_f)
    rowmask = rowi < num_pos
    lose = jnp.sum(accL[...], axis=1, keepdims=True) > 0.0
    win_ref[...] = jnp.where(rowmask & (~lose), 1.0, 0.0)
    part_S = jnp.sum(jnp.where(rowmask, S, 0.0))
    part_a = jnp.sum(jnp.where(rowmask, ar, 0.0))

    @pl.when(r == 0)
    def _init():
        smacc[0] = part_S
        smacc[1] = part_a

    @pl.when(r > 0)
    def _accum():
        smacc[0] = smacc[0] + part_S
        smacc[1] = smacc[1] + part_a

    @pl.when(r == NBR - 1)
    def _loss():
        val = (LAM / RHO) * smacc[0] / (np_f * nn_f) + smacc[1] / np_f
        loss_ref[...] = jnp.reshape(val, (1, 1))


def _tc_dense(cnt, ppA, apA, kpA, ppB, apB, kpB, pnA, pnB):
    col_spec = pl.BlockSpec((R_BLK, 1), lambda r: (r, 0))
    row_spec = pl.BlockSpec((1, B), lambda r: (0, 0))
    c2 = lambda x: x.reshape(B, 1)
    r2 = lambda x: x.reshape(1, B)
    return pl.pallas_call(
        _tc_dense_body,
        grid=(NBR,),
        in_specs=[pl.BlockSpec(memory_space=pltpu.SMEM),
                  col_spec, col_spec, col_spec, col_spec, col_spec, col_spec,
                  row_spec, row_spec, row_spec, row_spec],
        out_specs=[pl.BlockSpec((R_BLK, 1), lambda r: (r, 0)),
                   pl.BlockSpec((R_BLK, 1), lambda r: (r, 0)),
                   pl.BlockSpec((R_BLK, 1), lambda r: (r, 0)),
                   pl.BlockSpec((1, 1), lambda r: (0, 0))],
        out_shape=[jax.ShapeDtypeStruct((B, 1), jnp.float32),
                   jax.ShapeDtypeStruct((B, 1), jnp.float32),
                   jax.ShapeDtypeStruct((B, 1), jnp.int32),
                   jax.ShapeDtypeStruct((1, 1), jnp.float32)],
        scratch_shapes=[pltpu.VMEM((R_BLK, 128), jnp.float32),
                        pltpu.VMEM((R_BLK, 128), jnp.float32),
                        pltpu.VMEM((R_BLK, 128), jnp.float32),
                        pltpu.SMEM((2,), jnp.float32)],
    )(cnt, c2(ppA), c2(apA), c2(kpA), c2(ppB), c2(apB), c2(kpB),
      r2(pnA), r2(kpA), r2(pnB), r2(kpB))


def _sc_scatter_body(alpha_hbm, key_hbm, val_hbm, win_hbm, out_hbm,
                     tbl_v, key_v, val_v, win_v):
    wid = lax.axis_index("s") * NC + lax.axis_index("c")
    lo = wid * CHUNK
    pltpu.sync_copy(alpha_hbm.at[pl.ds(lo, CHUNK)], tbl_v)
    pltpu.sync_copy(key_hbm, key_v)
    pltpu.sync_copy(val_hbm, val_v)
    pltpu.sync_copy(win_hbm, win_v)

    def body(k, carry):
        off = pl.multiple_of(k * 16, 16)
        iv = key_v[pl.ds(off, 16)] >> 13
        vv = val_v[pl.ds(off, 16)]
        wv = win_v[pl.ds(off, 16)]
        msk = (iv >= lo) & (iv < lo + CHUNK) & (wv > 0.0)
        plsc.store_scatter(tbl_v, [iv - lo], vv, mask=msk)
        return carry

    lax.fori_loop(0, B // 16, body, 0, unroll=8)
    pltpu.sync_copy(tbl_v, out_hbm.at[pl.ds(lo, CHUNK)])


def _sc_scatter(alpha_pad, keys, vals, win):
    return pl.kernel(
        _sc_scatter_body,
        out_type=jax.ShapeDtypeStruct((PAD_LEN,), jnp.float32),
        mesh=plsc.VectorSubcoreMesh(**_MESH),
        scratch_types=[
            pltpu.VMEM((CHUNK,), jnp.float32),
            pltpu.VMEM((B,), jnp.int32),
            pltpu.VMEM((B,), jnp.float32),
            pltpu.VMEM((B,), jnp.float32),
        ],
        compiler_params=pltpu.CompilerParams(needs_layout_passes=False),
    )(alpha_pad, keys, vals, win)


def kernel(y_pred, y_true, index, alpha):
    p = y_pred.reshape(B)
    yt = y_true.reshape(B)
    index = index.reshape(B).astype(jnp.int32)
    alpha_pad = jnp.pad(alpha.reshape(DATA_LEN), (0, PAD_LEN - DATA_LEN))

    slot = jnp.arange(B, dtype=jnp.int32)
    key = index * 8192 + slot
    psc = p * SQRT_LOG2E
    psr = (p + MARGIN) * SQRT_LOG2E

    (ppA, apA, kpA, pnA, ppB, apB, kpB, pnB, cnt) = _sc_compact(
        yt, psc, psr, key, index, alpha_pad)

    a_new, win, kout, loss = _tc_dense(cnt, ppA, apA, kpA, ppB, apB, kpB,
                                       pnA, pnB)

    alpha_out = _sc_scatter(alpha_pad, kout.reshape(B), a_new.reshape(B),
                            win.reshape(B))
    return loss.reshape(()), alpha_out[:DATA_LEN].reshape(DATA_LEN, 1)


# speedup vs baseline: 1.8418x; 1.0061x over previous
import functools
import math

import jax
import jax.numpy as jnp
from jax import lax
from jax.experimental import pallas as pl
from jax.experimental.pallas import tpu as pltpu
from jax.experimental.pallas import tpu_sc as plsc

DATA_LEN = 100000
RHO = 0.001
LR_DUAL = 0.001
MARGIN = 1.0
LAM = 1.0
LOG_RHO = math.log(RHO)

B = 4096
NC, NS = 2, 16
NW = NC * NS
PER_W = B // NW
CHUNK = 3136
PAD_LEN = NW * CHUNK

R_BLK = 512
NBR = B // R_BLK
C_BLK = 1024
NCB = B // C_BLK

LOG2E = 1.4426950408889634
LN2 = 0.6931471805599453
SQRT_LOG2E = LOG2E ** 0.5
KEY_PAD = -(2 ** 30)

_MESH = dict(core_axis_name="c", subcore_axis_name="s", num_cores=NC,
             num_subcores=NS)


def _sc_compact_body(yt_hbm, psc_hbm, psr_hbm, key_hbm, idx_hbm, alpha_hbm,
                     ppA_hbm, apA_hbm, kpA_hbm, pnA_hbm,
                     ppB_hbm, apB_hbm, kpB_hbm, pnB_hbm, cnt_hbm,
                     ytf_v, psc_v, psr_v, key_v, idx_v, a_v,
                     tgtp_v, tgtn_v, cnt_v, pp_s, ap_s, kp_s, pn_s, sem):
    cid = lax.axis_index("c")
    sid = lax.axis_index("s")
    wid = cid * NS + sid
    base = wid * PER_W
    d_yt = pltpu.async_copy(yt_hbm, ytf_v, sem)
    d_psc = pltpu.async_copy(psc_hbm.at[pl.ds(base, PER_W)], psc_v, sem)
    d_psr = pltpu.async_copy(psr_hbm.at[pl.ds(base, PER_W)], psr_v, sem)
    d_key = pltpu.async_copy(key_hbm.at[pl.ds(base, PER_W)], key_v, sem)
    d_idx = pltpu.async_copy(idx_hbm.at[pl.ds(base, PER_W)], idx_v, sem)
    d_idx.wait()
    d_ga = pltpu.async_copy(alpha_hbm.at[idx_v], a_v, sem)
    d_yt.wait()

    def pbody(k, s16):
        return s16 + ytf_v[pl.ds(k * 16, 16)]

    acc16 = lax.fori_loop(0, wid * 8, pbody, jnp.zeros((16,), jnp.float32))
    npos_before = jnp.sum(acc16).astype(jnp.int32)
    nneg_before = base - npos_before

    lanes = lax.iota(jnp.int32, 16)
    cpos = jnp.int32(0)
    cneg = jnp.int32(0)
    for v in range(PER_W // 16):
        yv = ytf_v[pl.ds(base + v * 16, 16)]
        posm = yv == 1.0
        pinc = posm.astype(jnp.int32)
        cs = plsc.cumsum(pinc)
        padbase = B + base + v * 16
        tgt_p = jnp.where(posm, npos_before + cpos + cs - 1, padbase + lanes)
        tgt_n = jnp.where(posm, padbase + lanes,
                          nneg_before + cneg + (lanes + 1 - cs) - 1)
        tgtp_v[pl.ds(v * 16, 16)] = tgt_p
        tgtn_v[pl.ds(v * 16, 16)] = tgt_n
        nposv = jnp.sum(pinc)
        cpos = cpos + nposv
        cneg = cneg + (16 - nposv)

    d_psc.wait()
    d_psr.wait()
    d_key.wait()
    d_ga.wait()
    d_pp = pltpu.async_copy(psc_v, pp_s.at[tgtp_v], sem)
    d_ap = pltpu.async_copy(a_v, ap_s.at[tgtp_v], sem)
    d_kp = pltpu.async_copy(key_v, kp_s.at[tgtp_v], sem)
    d_pn = pltpu.async_copy(psr_v, pn_s.at[tgtn_v], sem)
    d_pp.wait()
    d_ap.wait()
    d_kp.wait()
    d_pn.wait()
    plsc.subcore_barrier()

    @pl.when(sid == 0)
    def _flush():
        @pl.when(cid == 0)
        def _a():
            pltpu.sync_copy(pp_s.at[pl.ds(0, B)], ppA_hbm)
            pltpu.sync_copy(ap_s.at[pl.ds(0, B)], apA_hbm)
            pltpu.sync_copy(kp_s.at[pl.ds(0, B)], kpA_hbm)
            pltpu.sync_copy(pn_s.at[pl.ds(0, B)], pnA_hbm)

        @pl.when(cid == 1)
        def _b():
            pltpu.sync_copy(pp_s.at[pl.ds(0, B)], ppB_hbm)
            pltpu.sync_copy(ap_s.at[pl.ds(0, B)], apB_hbm)
            pltpu.sync_copy(kp_s.at[pl.ds(0, B)], kpB_hbm)
            pltpu.sync_copy(pn_s.at[pl.ds(0, B)], pnB_hbm)

    @pl.when(sid == NS - 1)
    def _write_counts():
        tot_pos = npos_before + cpos

        @pl.when(cid == 1)
        def _tot():
            cnt_v[...] = jnp.where(lanes == 0, tot_pos,
                                   jnp.where(lanes == 1, B - tot_pos, 0))
            pltpu.sync_copy(cnt_v.at[pl.ds(0, 8)], cnt_hbm.at[pl.ds(0, 8)])

        @pl.when(cid == 0)
        def _fh():
            cnt_v[...] = jnp.where(lanes == 0, tot_pos,
                                   jnp.where(lanes == 1, B // 2 - tot_pos, 0))
            pltpu.sync_copy(cnt_v.at[pl.ds(0, 8)], cnt_hbm.at[pl.ds(8, 8)])


def _sc_compact(yt, psc, psr, key, idx, alpha_pad):
    f32 = jnp.float32
    arr = lambda dt: jax.ShapeDtypeStruct((B,), dt)
    return pl.kernel(
        _sc_compact_body,
        out_type=(arr(f32), arr(f32), arr(jnp.int32), arr(f32),
                  arr(f32), arr(f32), arr(jnp.int32), arr(f32),
                  jax.ShapeDtypeStruct((16,), jnp.int32)),
        mesh=plsc.VectorSubcoreMesh(**_MESH),
        scratch_types=[
            pltpu.VMEM((B,), f32),
            pltpu.VMEM((PER_W,), f32),
            pltpu.VMEM((PER_W,), f32),
            pltpu.VMEM((PER_W,), jnp.int32),
            pltpu.VMEM((PER_W,), jnp.int32),
            pltpu.VMEM((PER_W,), f32),
            pltpu.VMEM((PER_W,), jnp.int32),
            pltpu.VMEM((PER_W,), jnp.int32),
            pltpu.VMEM((16,), jnp.int32),
            pltpu.VMEM_SHARED((2 * B,), f32),
            pltpu.VMEM_SHARED((2 * B,), f32),
            pltpu.VMEM_SHARED((2 * B,), jnp.int32),
            pltpu.VMEM_SHARED((2 * B,), f32),
            pltpu.SemaphoreType.DMA,
        ],
        compiler_params=pltpu.CompilerParams(needs_layout_passes=False),
    )(yt, psc, psr, key, idx, alpha_pad)


def _main_chunk(cc, lanec_g, pnA_row, pnB_row, pr, cr2, nnfh, num_neg,
                accS, accT):
    pc = jnp.where(lanec_g < nnfh,
                   pnA_row[:, cc * C_BLK:(cc + 1) * C_BLK],
                   pnB_row[:, cc * C_BLK:(cc + 1) * C_BLK])
    negmask = lanec_g < num_neg
    h = jnp.maximum(pc - pr, 0.0)
    e2 = jnp.where(negmask, h * h + cr2, -jnp.inf)
    nabs = lax.bitcast_convert_type(
        lax.bitcast_convert_type(e2, jnp.int32) | jnp.int32(-2147483648),
        jnp.float32)
    u = jnp.exp2(nabs)
    l2 = jnp.log2(1.0 + u)
    m2 = jnp.maximum(e2, 0.0)
    th = jnp.tanh(e2 * (LN2 * 0.5))
    accS[...] += _lanefold(m2 + l2)
    accT[...] += _lanefold(th)


def _dup_chunk(cc, lanec_g, kpA_row, kpB_row, kr, npfh, num_pos, accL):
    kc0 = jnp.where(lanec_g < npfh,
                    kpA_row[:, cc * C_BLK:(cc + 1) * C_BLK],
                    kpB_row[:, cc * C_BLK:(cc + 1) * C_BLK])
    kc = jnp.where(lanec_g < num_pos, kc0, jnp.int32(KEY_PAD))
    delta = lax.bitcast_convert_type(kc - kr - 1, jnp.uint32)
    hit = (delta < jnp.uint32(4095)).astype(jnp.float32)
    accL[...] += _lanefold(hit)


def _lanefold(x):
    out = x[:, 0:128]
    for q in range(1, C_BLK // 128):
        out = out + x[:, q * 128:(q + 1) * 128]
    return out


def _tc_dense_body(cnt, ppA_col, apA_col, kpA_col, ppB_col, apB_col, kpB_col,
                   pnA_row, kpA_row, pnB_row, kpB_row,
                   a_new_ref, win_ref, kout_ref, loss_ref,
                   accS, accT, accL, smacc):
    r = pl.program_id(0)
    num_pos = cnt[0]
    num_neg = cnt[1]
    npfh = cnt[8]
    nnfh = cnt[9]

    accS[...] = jnp.zeros((R_BLK, 128), jnp.float32)
    accT[...] = jnp.zeros((R_BLK, 128), jnp.float32)
    accL[...] = jnp.zeros((R_BLK, 128), jnp.float32)

    row_active = r * R_BLK < num_pos
    lanec = lax.broadcasted_iota(jnp.int32, (1, C_BLK), 1)
    rowi = lax.broadcasted_iota(jnp.int32, (R_BLK, 1), 0) + r * R_BLK
    rowselA = rowi < npfh
    pr = jnp.where(rowselA, ppA_col[...], ppB_col[...])
    ar = jnp.where(rowselA, apA_col[...], apB_col[...])
    kr = jnp.where(rowselA, kpA_col[...], kpB_col[...])
    cr2 = (LOG_RHO - ar) * LOG2E
    kout_ref[...] = kr

    for cc in range(NCB):
        lanec_g = lanec + cc * C_BLK

        @pl.when(row_active & (cc * C_BLK < num_neg) & (cc * C_BLK < num_pos))
        def _both(cc=cc, lanec_g=lanec_g):
            _main_chunk(cc, lanec_g, pnA_row, pnB_row, pr, cr2, nnfh, num_neg,
                        accS, accT)
            _dup_chunk(cc, lanec_g, kpA_row, kpB_row, kr, npfh, num_pos, accL)

        @pl.when(row_active & (cc * C_BLK < num_neg) & (cc * C_BLK >= num_pos))
        def _main_only(cc=cc, lanec_g=lanec_g):
            _main_chunk(cc, lanec_g, pnA_row, pnB_row, pr, cr2, nnfh, num_neg,
                        accS, accT)

        @pl.when(row_active & (cc * C_BLK >= num_neg) & (cc * C_BLK < num_pos))
        def _dup_only(cc=cc, lanec_g=lanec_g):
            _dup_chunk(cc, lanec_g, kpA_row, kpB_row, kr, npfh, num_pos, accL)

    np_f = num_pos.astype(jnp.float32)
    nn_f = num_neg.astype(jnp.float32)
    nact = (num_neg + (C_BLK - 1)) // C_BLK
    W = 0.5 * ((nact * C_BLK).astype(jnp.float32)
               + jnp.sum(accT[...], axis=1, keepdims=True))
    S = LN2 * jnp.sum(accS[...], axis=1, keepdims=True)
    a_new_ref[...] = ar - LR_DUAL * (1.0 - W / nn_f)
    rowmask = rowi < num_pos
    lose = jnp.sum(accL[...], axis=1, keepdims=True) > 0.0
    win_ref[...] = jnp.where(rowmask & (~lose), 1.0, 0.0)
    part_S = jnp.sum(jnp.where(rowmask, S, 0.0))
    part_a = jnp.sum(jnp.where(rowmask, ar, 0.0))

    @pl.when(r == 0)
    def _init():
        smacc[0] = part_S
        smacc[1] = part_a

    @pl.when(r > 0)
    def _accum():
        smacc[0] = smacc[0] + part_S
        smacc[1] = smacc[1] + part_a

    @pl.when(r == NBR - 1)
    def _loss():
        val = (LAM / RHO) * smacc[0] / (np_f * nn_f) + smacc[1] / np_f
        loss_ref[...] = jnp.reshape(val, (1, 1))


def _tc_dense(cnt, ppA, apA, kpA, ppB, apB, kpB, pnA, pnB):
    col_spec = pl.BlockSpec((R_BLK, 1), lambda r: (r, 0))
    row_spec = pl.BlockSpec((1, B), lambda r: (0, 0))
    c2 = lambda x: x.reshape(B, 1)
    r2 = lambda x: x.reshape(1, B)
    return pl.pallas_call(
        _tc_dense_body,
        grid=(NBR,),
        in_specs=[pl.BlockSpec(memory_space=pltpu.SMEM),
                  col_spec, col_spec, col_spec, col_spec, col_spec, col_spec,
                  row_spec, row_spec, row_spec, row_spec],
        out_specs=[pl.BlockSpec((R_BLK, 1), lambda r: (r, 0)),
                   pl.BlockSpec((R_BLK, 1), lambda r: (r, 0)),
                   pl.BlockSpec((R_BLK, 1), lambda r: (r, 0)),
                   pl.BlockSpec((1, 1), lambda r: (0, 0))],
        out_shape=[jax.ShapeDtypeStruct((B, 1), jnp.float32),
                   jax.ShapeDtypeStruct((B, 1), jnp.float32),
                   jax.ShapeDtypeStruct((B, 1), jnp.int32),
                   jax.ShapeDtypeStruct((1, 1), jnp.float32)],
        scratch_shapes=[pltpu.VMEM((R_BLK, 128), jnp.float32),
                        pltpu.VMEM((R_BLK, 128), jnp.float32),
                        pltpu.VMEM((R_BLK, 128), jnp.float32),
                        pltpu.SMEM((2,), jnp.float32)],
    )(cnt, c2(ppA), c2(apA), c2(kpA), c2(ppB), c2(apB), c2(kpB),
      r2(pnA), r2(kpA), r2(pnB), r2(kpB))


def _sc_scatter_body(alpha_hbm, key_hbm, val_hbm, win_hbm, out_hbm,
                     tbl_v, key_v, val_v, win_v):
    wid = lax.axis_index("s") * NC + lax.axis_index("c")
    lo = wid * CHUNK
    pltpu.sync_copy(alpha_hbm.at[pl.ds(lo, CHUNK)], tbl_v)
    pltpu.sync_copy(key_hbm, key_v)
    pltpu.sync_copy(val_hbm, val_v)
    pltpu.sync_copy(win_hbm, win_v)

    def body(k, carry):
        off = pl.multiple_of(k * 16, 16)
        iv = key_v[pl.ds(off, 16)] >> 13
        vv = val_v[pl.ds(off, 16)]
        wv = win_v[pl.ds(off, 16)]
        msk = (iv >= lo) & (iv < lo + CHUNK) & (wv > 0.0)
        plsc.store_scatter(tbl_v, [iv - lo], vv, mask=msk)
        return carry

    lax.fori_loop(0, B // 16, body, 0, unroll=8)
    pltpu.sync_copy(tbl_v, out_hbm.at[pl.ds(lo, CHUNK)])


def _sc_scatter(alpha_pad, keys, vals, win):
    return pl.kernel(
        _sc_scatter_body,
        out_type=jax.ShapeDtypeStruct((PAD_LEN,), jnp.float32),
        mesh=plsc.VectorSubcoreMesh(**_MESH),
        scratch_types=[
            pltpu.VMEM((CHUNK,), jnp.float32),
            pltpu.VMEM((B,), jnp.int32),
            pltpu.VMEM((B,), jnp.float32),
            pltpu.VMEM((B,), jnp.float32),
        ],
        compiler_params=pltpu.CompilerParams(needs_layout_passes=False),
    )(alpha_pad, keys, vals, win)


def kernel(y_pred, y_true, index, alpha):
    p = y_pred.reshape(B)
    yt = y_true.reshape(B)
    index = index.reshape(B).astype(jnp.int32)
    alpha_pad = jnp.pad(alpha.reshape(DATA_LEN), (0, PAD_LEN - DATA_LEN))

    slot = jnp.arange(B, dtype=jnp.int32)
    key = index * 8192 + slot
    psc = p * SQRT_LOG2E
    psr = (p + MARGIN) * SQRT_LOG2E

    (ppA, apA, kpA, pnA, ppB, apB, kpB, pnB, cnt) = _sc_compact(
        yt, psc, psr, key, index, alpha_pad)

    a_new, win, kout, loss = _tc_dense(cnt, ppA, apA, kpA, ppB, apB, kpB,
                                       pnA, pnB)

    alpha_out = _sc_scatter(alpha_pad, kout.reshape(B), a_new.reshape(B),
                            win.reshape(B))
    return loss.reshape(()), alpha_out[:DATA_LEN].reshape(DATA_LEN, 1)
